# Initial kernel scaffold; baseline (speedup 1.0000x reference)
import jax
import jax.numpy as jnp
from jax.experimental import pallas as pl


def kernel(node_ids, edge_index, att, entity_table, W1_0, b1_0, W2_0, b2_0, W1_1, b1_1, W2_1, b2_1):
    raise NotImplementedError("write your pallas kernel here")


# speedup vs baseline: 4.6894x; 4.6894x over previous
import dataclasses
import functools

import jax
import jax.numpy as jnp
from jax import lax
from jax.experimental import pallas as pl
from jax.experimental.pallas import tpu as pltpu
from jax.experimental.pallas import tpu_sc as plsc

N_NODES = 10000
N_EDGES = 320000
NC = 2
NS = 16
NW = NC * NS
EDGES_PER_TILE = N_EDGES // NW
CHUNK = 200
NCHUNK = EDGES_PER_TILE // CHUNK
ROWS_PER_TILE = 624
ROWS_REM = N_NODES - NS * ROWS_PER_TILE


def _spmm_sc(table, src, dst, att, dim):
    mesh = plsc.VectorSubcoreMesh(core_axis_name="c", subcore_axis_name="s")

    def body(table_hbm, src_hbm, dst_hbm, att_hbm, out_hbm,
             src_v, dst_v, rows_v, att_v, acc_sh, sem):
        c = lax.axis_index("c")
        s = lax.axis_index("s")
        wid = c * NS + s

        zvec = jnp.zeros((16,), jnp.float32)

        @pl.loop(0, CHUNK)
        def _(e):
            for j in range(dim // 16):
                rows_v[e, pl.ds(j * 16, 16)] = zvec

        r0 = s * ROWS_PER_TILE
        nfull = ROWS_PER_TILE // CHUNK
        rem = ROWS_PER_TILE - nfull * CHUNK
        for i in range(nfull):
            pltpu.sync_copy(rows_v, acc_sh.at[pl.ds(r0 + i * CHUNK, CHUNK)])
        if rem:
            pltpu.sync_copy(rows_v.at[pl.ds(0, rem)],
                            acc_sh.at[pl.ds(r0 + nfull * CHUNK, rem)])

        @pl.when(s == NS - 1)
        def _():
            pltpu.sync_copy(rows_v.at[pl.ds(0, ROWS_REM)],
                            acc_sh.at[pl.ds(NS * ROWS_PER_TILE, ROWS_REM)])

        plsc.subcore_barrier()

        @pl.loop(0, NCHUNK)
        def _(k):
            base = wid * EDGES_PER_TILE + k * CHUNK
            pltpu.sync_copy(src_hbm.at[pl.ds(base, CHUNK)], src_v)
            pltpu.sync_copy(dst_hbm.at[pl.ds(base, CHUNK)], dst_v)
            pltpu.sync_copy(att_hbm.at[pl.ds(base, CHUNK)], att_v)
            pltpu.async_copy(table_hbm.at[src_v], rows_v, sem).wait()

            @pl.loop(0, CHUNK)
            def _(e):
                a = plsc.load_gather(att_v, [jnp.full((16,), e, jnp.int32)])
                for j in range(dim // 16):
                    sl = pl.ds(j * 16, 16)
                    rows_v[e, sl] = rows_v[e, sl] * a

            pltpu.sync_copy(rows_v, acc_sh.at[dst_v], add=True)

        plsc.subcore_barrier()
        pltpu.sync_copy(acc_sh.at[pl.ds(r0, ROWS_PER_TILE)],
                        out_hbm.at[c, pl.ds(r0, ROWS_PER_TILE)])

        @pl.when(s == NS - 1)
        def _():
            pltpu.sync_copy(acc_sh.at[pl.ds(NS * ROWS_PER_TILE, ROWS_REM)],
                            out_hbm.at[c, pl.ds(NS * ROWS_PER_TILE, ROWS_REM)])

    k = pl.kernel(
        body,
        out_type=jax.ShapeDtypeStruct((NC, N_NODES, dim), jnp.float32),
        mesh=mesh,
        scratch_types=[
            pltpu.VMEM((CHUNK,), jnp.int32),
            pltpu.VMEM((CHUNK,), jnp.int32),
            pltpu.VMEM((CHUNK, dim), jnp.float32),
            pltpu.VMEM((CHUNK,), jnp.float32),
            pltpu.VMEM_SHARED((N_NODES, dim), jnp.float32),
            pltpu.SemaphoreType.DMA,
        ],
        compiler_params=_sc_compiler_params(),
    )
    return k(table, src, dst, att)


def _sc_compiler_params():
    cp = pltpu.CompilerParams()
    if "needs_layout_passes" in pltpu.CompilerParams.__dataclass_fields__:
        cp = dataclasses.replace(cp, needs_layout_passes=False)
    return cp


def _dense_body(pad_ego_to, dout, din, ego_ref, p_ref, w1_ref, b1_ref, w2_ref,
                b2_ref, e_ref, y_ref):
    ego = ego_ref[...]
    nh = (p_ref[0] + p_ref[1])[:, :din]
    x1 = jnp.dot(ego + nh, w1_ref[...],
                 preferred_element_type=jnp.float32,
                 precision=lax.Precision.HIGHEST) + b1_ref[...]
    x2 = jnp.dot(ego * nh, w2_ref[...],
                 preferred_element_type=jnp.float32,
                 precision=lax.Precision.HIGHEST) + b2_ref[...]
    l1 = jnp.where(x1 >= 0, x1, 0.01 * x1)
    l2 = jnp.where(x2 >= 0, x2, 0.01 * x2)
    e = l1 + l2
    if pad_ego_to > dout:
        e_ref[...] = jnp.concatenate(
            [e, jnp.zeros((e.shape[0], pad_ego_to - dout), jnp.float32)], axis=1)
    else:
        e_ref[...] = e
    nrm = jnp.sqrt(jnp.sum(e * e, axis=1, keepdims=True))
    y_ref[...] = e / jnp.maximum(nrm, 1e-12)


def _dense_tc(ego, partials, W1, b1, W2, b2, pad_ego_to=None):
    n, din = ego.shape
    dout = W1.shape[1]
    pdim = partials.shape[2]
    pad = pad_ego_to if pad_ego_to is not None else dout
    r = 1000
    grid = (n // r,)
    return pl.pallas_call(
        functools.partial(_dense_body, pad, dout, din),
        grid=grid,
        in_specs=[
            pl.BlockSpec((r, din), lambda i: (i, 0)),
            pl.BlockSpec((NC, r, pdim), lambda i: (0, i, 0)),
            pl.BlockSpec((din, dout), lambda i: (0, 0)),
            pl.BlockSpec((1, dout), lambda i: (0, 0)),
            pl.BlockSpec((din, dout), lambda i: (0, 0)),
            pl.BlockSpec((1, dout), lambda i: (0, 0)),
        ],
        out_specs=[pl.BlockSpec((r, pad), lambda i: (i, 0)),
                   pl.BlockSpec((r, dout), lambda i: (i, 0))],
        out_shape=[jax.ShapeDtypeStruct((n, pad), jnp.float32),
                   jax.ShapeDtypeStruct((n, dout), jnp.float32)],
    )(ego, partials, W1, b1.reshape(1, -1), W2, b2.reshape(1, -1))


def kernel(node_ids, edge_index, att, entity_table,
           W1_0, b1_0, W2_0, b2_0, W1_1, b1_1, W2_1, b2_1):
    ego0 = jnp.take(entity_table, node_ids, axis=0)
    src = edge_index[0]
    dst = edge_index[1]
    p0 = _spmm_sc(ego0, src, dst, att, 128)
    ego1p, y1 = _dense_tc(ego0, p0, W1_0, b1_0, W2_0, b2_0, pad_ego_to=128)
    p1 = _spmm_sc(ego1p, src, dst, att, 128)
    ego1 = ego1p[:, :64]
    _, y2 = _dense_tc(ego1, p1, W1_1, b1_1, W2_1, b2_1)
    return jnp.concatenate([ego0, y1, y2], axis=1)
